# baseline (speedup 1.0000x reference)
import functools

import jax
import jax.numpy as jnp
from jax import lax
from jax.experimental import pallas as pl
from jax.experimental.pallas import tpu as pltpu
from jax.experimental.pallas import tpu_sc as plsc

_B = 64
_TMEL = 2048
_NCH = 80
_TSRC = 512
_BB = 4
_K = 32
_NG = (_B - _K) // _BB
_NW = 32
_NSTRIP = _NCH // 8
_SPW = _NSTRIP * _K // _NW
_WPB = _NW // _K
_STRIP = 8 * _TMEL
_VEC = 16


def _tc_body(mt, mp, mq, mm3, mm2, pt, pp, et, ep, dt, ldp, sv,
             o_s1, o_s2, o_pitch, o_energy, o_dur, o_nmel, o_nsrc, acc, a1, a2):
    i = pl.program_id(0)

    @pl.when(i == 0)
    def _init():
        mmv = mm2[...]
        svv = sv[...]
        log_dur_trg = jnp.log(dt[...].astype(jnp.float32) + 1.0)
        acc[2] = jnp.sum(jnp.square(pp[...] - pt[...]) * mmv)
        acc[3] = jnp.sum(jnp.square(ep[...] - et[...]) * mmv)
        acc[4] = jnp.sum(jnp.square(ldp[...] - log_dur_trg) * svv)
        acc[5] = jnp.sum(mmv)
        acc[6] = jnp.sum(svv)
        a1[...] = jnp.zeros_like(a1)
        a2[...] = jnp.zeros_like(a2)

    t = mt[...]
    m = mm3[...]
    a1[...] += jnp.abs(mp[...] - t) * m
    a2[...] += jnp.abs(mq[...] - t) * m

    @pl.when(i == _NG - 1)
    def _fin():
        o_s1[0] = jnp.sum(a1[...])
        o_s2[0] = jnp.sum(a2[...])
        o_pitch[0] = acc[2]
        o_energy[0] = acc[3]
        o_dur[0] = acc[4]
        o_nmel[0] = acc[5]
        o_nsrc[0] = acc[6]


_sc_mesh = plsc.VectorSubcoreMesh(core_axis_name="c", subcore_axis_name="s")


@functools.partial(
    pl.kernel,
    out_type=jax.ShapeDtypeStruct((2, _NW, _VEC), jnp.float32),
    mesh=_sc_mesh,
    scratch_types=[
        pltpu.VMEM((128, 128), jnp.float32),
        pltpu.VMEM((128, 128), jnp.float32),
        pltpu.VMEM((128, 128), jnp.float32),
        pltpu.VMEM((128, 128), jnp.float32),
        pltpu.VMEM((128, 128), jnp.float32),
        pltpu.VMEM((128, 128), jnp.float32),
        pltpu.VMEM((128, 128), jnp.float32),
        pltpu.VMEM((_VEC,), jnp.float32),
        pltpu.VMEM((_VEC,), jnp.float32),
        pltpu.SemaphoreType.DMA,
        pltpu.SemaphoreType.DMA,
    ],
)
def _sc_partial(mt, mp, mq, mk, out, tb0, tb1, pb0, pb1, qb0, qb1, mb,
                a1b, a2b, sem0, sem1):
    w = lax.axis_index("s") * 2 + lax.axis_index("c")
    batch = w // _WPB
    first = (w % _WPB) * _SPW
    tbs, pbs, qbs = (tb0, tb1), (pb0, pb1), (qb0, qb1)
    sems = (sem0, sem1)

    pltpu.sync_copy(mk.at[pl.ds(batch * 128, 128)], mb)

    def issue(s, slot):
        base = (batch * _NCH + 8 * (first + s)) * (_TMEL // 128)
        sem = sems[slot]
        return (
            pltpu.async_copy(mt.at[pl.ds(base, 128)], tbs[slot], sem),
            pltpu.async_copy(mp.at[pl.ds(base, 128)], pbs[slot], sem),
            pltpu.async_copy(mq.at[pl.ds(base, 128)], qbs[slot], sem),
        )

    acc1 = jnp.zeros((_VEC,), jnp.float32)
    acc2 = jnp.zeros((_VEC,), jnp.float32)
    cps = issue(0, 0)
    for s in range(_SPW):
        slot = s % 2
        nxt = issue(s + 1, 1 - slot) if s + 1 < _SPW else None
        for cp in cps:
            cp.wait()
        tb, pb, qb = tbs[slot], pbs[slot], qbs[slot]

        def body(r, carry):
            c1, c2 = carry
            for g in range(8):
                sl = pl.ds(g * _VEC, _VEC)
                t = tb[r, sl]
                m = mb[r, sl]
                c1 = c1 + jnp.abs(pb[r, sl] - t) * m
                c2 = c2 + jnp.abs(qb[r, sl] - t) * m
            return c1, c2

        acc1, acc2 = lax.fori_loop(0, 128, body, (acc1, acc2))
        cps = nxt

    a1b[...] = acc1
    a2b[...] = acc2
    pltpu.sync_copy(a1b, out.at[0, w])
    pltpu.sync_copy(a2b, out.at[1, w])


def kernel(mel_trg, dur_trg, pitch_trg, energy_trg, mel_pred,
           mel_postnet_pred, log_dur_pred, pitch_pred, energy_pred,
           src_mask, mel_mask):
    mt = jnp.transpose(mel_trg, (0, 2, 1))
    mp = jnp.transpose(mel_pred, (0, 2, 1))
    mq = jnp.transpose(mel_postnet_pred, (0, 2, 1))
    mm2 = mel_mask.astype(jnp.float32)
    mm3 = mm2.reshape(_B, 1, _TMEL)
    sv = jnp.logical_not(src_mask).astype(jnp.float32)

    mt1 = mt.reshape(-1, 128)
    mp1 = mp.reshape(-1, 128)
    mq1 = mq.reshape(-1, 128)
    mk1 = jnp.broadcast_to(
        mm2.reshape(_B, _TMEL // 128, 1, 128), (_B, _TMEL // 128, 8, 128)
    ).reshape(-1, 128)

    sc_out = _sc_partial(mt1, mp1, mq1, mk1)

    mel_spec = pl.BlockSpec((_BB, _NCH, _TMEL), lambda i: (i + _K // _BB, 0, 0))
    full = lambda shape: pl.BlockSpec(shape, lambda i: (0,) * len(shape))
    out_spec = pl.BlockSpec(memory_space=pltpu.SMEM)
    outs = pl.pallas_call(
        _tc_body,
        grid=(_NG,),
        in_specs=[
            mel_spec,
            mel_spec,
            mel_spec,
            pl.BlockSpec((_BB, 1, _TMEL), lambda i: (i + _K // _BB, 0, 0)),
            full((_B, _TMEL)),
            full((_B, _TMEL)),
            full((_B, _TMEL)),
            full((_B, _TMEL)),
            full((_B, _TMEL)),
            full((_B, _TSRC)),
            full((_B, _TSRC)),
            full((_B, _TSRC)),
        ],
        out_specs=[out_spec] * 7,
        out_shape=[jax.ShapeDtypeStruct((1,), jnp.float32)] * 7,
        scratch_shapes=[pltpu.SMEM((8,), jnp.float32),
                        pltpu.VMEM((_BB, _NCH, _TMEL), jnp.float32),
                        pltpu.VMEM((_BB, _NCH, _TMEL), jnp.float32)],
    )(mt, mp, mq, mm3, mm2, pitch_trg, pitch_pred, energy_trg, energy_pred,
      dur_trg, log_dur_pred, sv)

    s1 = outs[0][0] + jnp.sum(sc_out[0])
    s2 = outs[1][0] + jnp.sum(sc_out[1])
    n_mel = outs[5][0]
    n_src = outs[6][0]
    mel_loss = s1 / (n_mel * _NCH)
    post_loss = s2 / (n_mel * _NCH)
    pitch_loss = outs[2][0] / n_mel
    energy_loss = outs[3][0] / n_mel
    dur_loss = outs[4][0] / n_src
    total = mel_loss + post_loss + dur_loss + pitch_loss + energy_loss
    return (total, mel_loss, post_loss, dur_loss, pitch_loss, energy_loss)


# speedup vs baseline: 2.8697x; 2.8697x over previous
import functools

import jax
import jax.numpy as jnp
from jax import lax
from jax.experimental import pallas as pl
from jax.experimental.pallas import tpu as pltpu
from jax.experimental.pallas import tpu_sc as plsc

_B = 64
_TMEL = 2048
_NCH = 80
_TSRC = 512
_BB = 4
_NG = _B // _BB
_NW = 32
_RPW = _B // _NW
_VEC = 16


def _tc_body(mt, mp, mq, mm3, dt, ldp, sv,
             o_s1, o_s2, o_dur, o_nsrc, acc, a1, a2):
    i = pl.program_id(0)

    @pl.when(i == 0)
    def _init():
        svv = sv[...]
        log_dur_trg = jnp.log(dt[...].astype(jnp.float32) + 1.0)
        acc[0] = jnp.sum(jnp.square(ldp[...] - log_dur_trg) * svv)
        acc[1] = jnp.sum(svv)
        a1[...] = jnp.zeros_like(a1)
        a2[...] = jnp.zeros_like(a2)

    t = mt[...]
    m = mm3[...]
    a1[...] += jnp.abs(mp[...] - t) * m
    a2[...] += jnp.abs(mq[...] - t) * m

    @pl.when(i == _NG - 1)
    def _fin():
        o_s1[0] = jnp.sum(a1[...])
        o_s2[0] = jnp.sum(a2[...])
        o_dur[0] = acc[0]
        o_nsrc[0] = acc[1]


_sc_mesh = plsc.VectorSubcoreMesh(core_axis_name="c", subcore_axis_name="s")


@functools.partial(
    pl.kernel,
    out_type=jax.ShapeDtypeStruct((3, _NW, _VEC), jnp.float32),
    mesh=_sc_mesh,
    scratch_types=[
        pltpu.VMEM((_RPW, _TMEL), jnp.float32),
        pltpu.VMEM((_RPW, _TMEL), jnp.float32),
        pltpu.VMEM((_RPW, _TMEL), jnp.float32),
        pltpu.VMEM((_RPW, _TMEL), jnp.float32),
        pltpu.VMEM((_RPW, _TMEL), jnp.float32),
        pltpu.VMEM((_VEC,), jnp.float32),
        pltpu.VMEM((_VEC,), jnp.float32),
        pltpu.VMEM((_VEC,), jnp.float32),
        pltpu.SemaphoreType.DMA,
    ],
)
def _sc_small(pt, pp, et, ep, mk, out, ptb, ppb, etb, epb, mkb,
              sp, se, sn, sem):
    w = lax.axis_index("s") * 2 + lax.axis_index("c")
    row = w * _RPW
    cps = (
        pltpu.async_copy(pt.at[pl.ds(row, _RPW), :], ptb, sem),
        pltpu.async_copy(pp.at[pl.ds(row, _RPW), :], ppb, sem),
        pltpu.async_copy(et.at[pl.ds(row, _RPW), :], etb, sem),
        pltpu.async_copy(ep.at[pl.ds(row, _RPW), :], epb, sem),
        pltpu.async_copy(mk.at[pl.ds(row, _RPW), :], mkb, sem),
    )
    for cp in cps:
        cp.wait()

    accp = jnp.zeros((_VEC,), jnp.float32)
    acce = jnp.zeros((_VEC,), jnp.float32)
    accn = jnp.zeros((_VEC,), jnp.float32)

    def body(j, carry):
        cp_, ce_, cn_ = carry
        for r in range(_RPW):
            sl = pl.ds(j * _VEC, _VEC)
            m = mkb[r, sl]
            dp = ppb[r, sl] - ptb[r, sl]
            de = epb[r, sl] - etb[r, sl]
            cp_ = cp_ + dp * dp * m
            ce_ = ce_ + de * de * m
            cn_ = cn_ + m
        return cp_, ce_, cn_

    accp, acce, accn = lax.fori_loop(0, _TMEL // _VEC, body, (accp, acce, accn))
    sp[...] = accp
    se[...] = acce
    sn[...] = accn
    pltpu.sync_copy(sp, out.at[0, w])
    pltpu.sync_copy(se, out.at[1, w])
    pltpu.sync_copy(sn, out.at[2, w])


def kernel(mel_trg, dur_trg, pitch_trg, energy_trg, mel_pred,
           mel_postnet_pred, log_dur_pred, pitch_pred, energy_pred,
           src_mask, mel_mask):
    mt = jnp.transpose(mel_trg, (0, 2, 1))
    mp = jnp.transpose(mel_pred, (0, 2, 1))
    mq = jnp.transpose(mel_postnet_pred, (0, 2, 1))
    mm2 = mel_mask.astype(jnp.float32)
    mm3 = mm2.reshape(_B, 1, _TMEL)
    sv = jnp.logical_not(src_mask).astype(jnp.float32)

    sc_out = _sc_small(pitch_trg, pitch_pred, energy_trg, energy_pred, mm2)

    mel_spec = pl.BlockSpec((_BB, _NCH, _TMEL), lambda i: (i, 0, 0))
    full = lambda shape: pl.BlockSpec(shape, lambda i: (0,) * len(shape))
    out_spec = pl.BlockSpec(memory_space=pltpu.SMEM)
    outs = pl.pallas_call(
        _tc_body,
        grid=(_NG,),
        in_specs=[
            mel_spec,
            mel_spec,
            mel_spec,
            pl.BlockSpec((_BB, 1, _TMEL), lambda i: (i, 0, 0)),
            full((_B, _TSRC)),
            full((_B, _TSRC)),
            full((_B, _TSRC)),
        ],
        out_specs=[out_spec] * 4,
        out_shape=[jax.ShapeDtypeStruct((1,), jnp.float32)] * 4,
        scratch_shapes=[pltpu.SMEM((2,), jnp.float32),
                        pltpu.VMEM((_BB, _NCH, _TMEL), jnp.float32),
                        pltpu.VMEM((_BB, _NCH, _TMEL), jnp.float32)],
    )(mt, mp, mq, mm3, dur_trg, log_dur_pred, sv)

    n_mel = jnp.sum(sc_out[2])
    n_src = outs[3][0]
    mel_loss = outs[0][0] / (n_mel * _NCH)
    post_loss = outs[1][0] / (n_mel * _NCH)
    pitch_loss = jnp.sum(sc_out[0]) / n_mel
    energy_loss = jnp.sum(sc_out[1]) / n_mel
    dur_loss = outs[2][0] / n_src
    total = mel_loss + post_loss + dur_loss + pitch_loss + energy_loss
    return (total, mel_loss, post_loss, dur_loss, pitch_loss, energy_loss)


# speedup vs baseline: 4.4574x; 1.5533x over previous
import jax
import jax.numpy as jnp
from jax.experimental import pallas as pl
from jax.experimental.pallas import tpu as pltpu

_B = 64
_TMEL = 2048
_NCH = 80
_TSRC = 512
_BB = 8
_NG = _B // _BB


def _loss_body(mt, mp, mq, mm3, mm2, pt, pp, et, ep, dt, ldp, sv,
               o_total, o_mel, o_post, o_dur, o_pitch, o_energy, acc, a1, a2):
    i = pl.program_id(0)

    @pl.when(i == 0)
    def _init():
        mmv = mm2[...]
        svv = sv[...]
        log_dur_trg = jnp.log(dt[...].astype(jnp.float32) + 1.0)
        acc[2] = jnp.sum(jnp.square(pp[...] - pt[...]) * mmv)
        acc[3] = jnp.sum(jnp.square(ep[...] - et[...]) * mmv)
        acc[4] = jnp.sum(jnp.square(ldp[...] - log_dur_trg) * svv)
        acc[5] = jnp.sum(mmv)
        acc[6] = jnp.sum(svv)
        a1[...] = jnp.zeros_like(a1)
        a2[...] = jnp.zeros_like(a2)

    t = mt[...]
    m = mm3[...]
    a1[...] += jnp.abs(mp[...] - t) * m
    a2[...] += jnp.abs(mq[...] - t) * m

    @pl.when(i == _NG - 1)
    def _fin():
        n_mel = acc[5]
        n_src = acc[6]
        mel_loss = jnp.sum(a1[...]) / (n_mel * _NCH)
        post_loss = jnp.sum(a2[...]) / (n_mel * _NCH)
        pitch_loss = acc[2] / n_mel
        energy_loss = acc[3] / n_mel
        dur_loss = acc[4] / n_src
        o_mel[0] = mel_loss
        o_post[0] = post_loss
        o_dur[0] = dur_loss
        o_pitch[0] = pitch_loss
        o_energy[0] = energy_loss
        o_total[0] = mel_loss + post_loss + dur_loss + pitch_loss + energy_loss


def kernel(mel_trg, dur_trg, pitch_trg, energy_trg, mel_pred,
           mel_postnet_pred, log_dur_pred, pitch_pred, energy_pred,
           src_mask, mel_mask):
    mt = jnp.transpose(mel_trg, (0, 2, 1))
    mp = jnp.transpose(mel_pred, (0, 2, 1))
    mq = jnp.transpose(mel_postnet_pred, (0, 2, 1))
    mm2 = mel_mask.astype(jnp.float32)
    mm3 = mm2.reshape(_B, 1, _TMEL)
    sv = jnp.logical_not(src_mask).astype(jnp.float32)

    mel_spec = pl.BlockSpec((_BB, _NCH, _TMEL), lambda i: (i, 0, 0))
    full = lambda shape: pl.BlockSpec(shape, lambda i: (0,) * len(shape))
    out_spec = pl.BlockSpec(memory_space=pltpu.SMEM)
    outs = pl.pallas_call(
        _loss_body,
        grid=(_NG,),
        in_specs=[
            mel_spec,
            mel_spec,
            mel_spec,
            pl.BlockSpec((_BB, 1, _TMEL), lambda i: (i, 0, 0)),
            full((_B, _TMEL)),
            full((_B, _TMEL)),
            full((_B, _TMEL)),
            full((_B, _TMEL)),
            full((_B, _TMEL)),
            full((_B, _TSRC)),
            full((_B, _TSRC)),
            full((_B, _TSRC)),
        ],
        out_specs=[out_spec] * 6,
        out_shape=[jax.ShapeDtypeStruct((1,), jnp.float32)] * 6,
        scratch_shapes=[pltpu.SMEM((8,), jnp.float32),
                        pltpu.VMEM((_BB, _NCH, _TMEL), jnp.float32),
                        pltpu.VMEM((_BB, _NCH, _TMEL), jnp.float32)],
    )(mt, mp, mq, mm3, mm2, pitch_trg, pitch_pred, energy_trg, energy_pred,
      dur_trg, log_dur_pred, sv)

    total, mel, post, dur, pitch, energy = [o[0] for o in outs]
    return (total, mel, post, dur, pitch, energy)
